# baseline (speedup 1.0000x reference)
import functools

import jax
import jax.numpy as jnp
import numpy as np
from jax import lax
from jax.experimental import pallas as pl
from jax.experimental.pallas import tpu as pltpu
from jax.experimental.pallas import tpu_sc as plsc

N = 10000
E = 320000
D = 128

NC = 2
NS = 16
NW = NC * NS

CHUNK = 128
CHUNKS = 80
HALF = CHUNKS // 2
EPW = CHUNKS * CHUNK
E_PAD = EPW * NW

NPAD = 10240
RPS = NPAD // NS

RE = E // CHUNK
PAD_ROWS = NW * CHUNKS - RE

f32 = jnp.float32

_pi = np.arange(PAD_ROWS * CHUNK, dtype=np.int32)
_PAD_NP = np.stack([(_pi * 131) % N, N + (_pi % (NPAD - N))]).reshape(2, PAD_ROWS, CHUNK)


def _zero_vmem_2d(ref, rows, cols):
    assert cols % 16 == 0
    z = jnp.zeros((16,), f32)

    def body(r, _):
        for k in range(cols // 16):
            ref[r, pl.ds(16 * k, 16)] = z
        return 0

    lax.fori_loop(0, rows, body, 0)


@functools.partial(
    pl.kernel,
    out_type=jax.ShapeDtypeStruct((NC, NPAD), f32),
    mesh=plsc.VectorSubcoreMesh(core_axis_name="c", subcore_axis_name="s"),
    scratch_types=[
        pltpu.VMEM((CHUNKS, CHUNK), jnp.int32),
        pltpu.VMEM((CHUNK,), f32),
        pltpu.VMEM((CHUNK,), f32),
        pltpu.VMEM_SHARED((NPAD,), f32),
    ],
)
def _deg_kernel(ei_hbm, out_hbm, idx_v, ones_v, zeros_v, acc):
    c = lax.axis_index("c")
    s = lax.axis_index("s")
    wid = s * NC + c

    pltpu.sync_copy(ei_hbm.at[1, pl.ds(wid * CHUNKS, CHUNKS)], idx_v)

    one = jnp.ones((16,), f32)
    zero = jnp.zeros((16,), f32)
    for r in range(CHUNK // 16):
        ones_v[pl.ds(16 * r, 16)] = one
        zeros_v[pl.ds(16 * r, 16)] = zero

    base = s * RPS
    for k in range(RPS // CHUNK):
        pltpu.sync_copy(zeros_v, acc.at[pl.ds(base + k * CHUNK, CHUNK)])

    plsc.subcore_barrier()

    def step(j, _):
        pltpu.sync_copy(ones_v, acc.at[idx_v.at[j]], add=True)
        return 0

    lax.fori_loop(0, CHUNKS, step, 0)

    plsc.subcore_barrier()
    pltpu.sync_copy(acc.at[pl.ds(base, RPS)], out_hbm.at[c, pl.ds(base, RPS)])


@functools.partial(
    pl.kernel,
    out_type=jax.ShapeDtypeStruct((NC, NPAD, D), f32),
    mesh=plsc.VectorSubcoreMesh(core_axis_name="c", subcore_axis_name="s"),
    scratch_types=[
        pltpu.VMEM((HALF, CHUNK), jnp.int32),
        pltpu.VMEM((HALF, CHUNK), jnp.int32),
        pltpu.VMEM((CHUNK, D), f32),
        pltpu.VMEM((CHUNK, D), f32),
        pltpu.VMEM_SHARED((NPAD, D), f32),
        pltpu.SemaphoreType.DMA,
        pltpu.SemaphoreType.DMA,
        pltpu.SemaphoreType.DMA,
        pltpu.SemaphoreType.DMA,
    ],
)
def _scatter_kernel(t_hbm, ei_hbm, out_hbm,
                    src_v, dst_v, rows0_v, rows1_v, acc,
                    sem0, sem1, ssem0, ssem1):
    c = lax.axis_index("c")
    s = lax.axis_index("s")
    wid = s * NC + c

    def stage(g):
        row0 = wid * CHUNKS + g * HALF
        pltpu.sync_copy(ei_hbm.at[0, pl.ds(row0, HALF)], src_v)
        pltpu.sync_copy(ei_hbm.at[1, pl.ds(row0, HALF)], dst_v)

    stage(0)
    pltpu.async_copy(t_hbm.at[src_v.at[0]], rows0_v, sem0)

    _zero_vmem_2d(rows1_v, CHUNK, D)

    base = s * RPS
    for k in range(RPS // CHUNK):
        pltpu.sync_copy(rows1_v, acc.at[pl.ds(base + k * CHUNK, CHUNK)])

    plsc.subcore_barrier()
    pltpu.async_copy(t_hbm.at[src_v.at[1]], rows1_v, sem1)

    for g in range(2):
        if g:
            stage(1)
            pltpu.async_copy(t_hbm.at[src_v.at[0]], rows0_v, sem0)
            pltpu.async_copy(t_hbm.at[src_v.at[1]], rows1_v, sem1)

        def step2(jj, _):
            j0 = 2 * jj
            j1 = j0 + 1

            pltpu.make_async_copy(t_hbm.at[src_v.at[j0]], rows0_v, sem0).wait()
            d0 = pltpu.async_copy(rows0_v, acc.at[dst_v.at[j0]], ssem0, add=True)

            pltpu.make_async_copy(t_hbm.at[src_v.at[j1]], rows1_v, sem1).wait()
            d1 = pltpu.async_copy(rows1_v, acc.at[dst_v.at[j1]], ssem1, add=True)

            d0.wait()

            @pl.when(j0 + 2 < HALF)
            def _():
                pltpu.async_copy(t_hbm.at[src_v.at[j0 + 2]], rows0_v, sem0)

            d1.wait()

            @pl.when(j1 + 2 < HALF)
            def _():
                pltpu.async_copy(t_hbm.at[src_v.at[j1 + 2]], rows1_v, sem1)

            return 0

        lax.fori_loop(0, HALF // 2, step2, 0)

    plsc.subcore_barrier()
    pltpu.sync_copy(acc.at[pl.ds(base, RPS)], out_hbm.at[c, pl.ds(base, RPS)])


BLK = 2560
GRID = -(-N // BLK)


def _dinv(d):
    return lax.rsqrt(jnp.maximum(d, 1.0))


def _tc1_body(x_ref, w_ref, d_ref, o_ref):
    dinv = _dinv(d_ref[...])
    o_ref[...] = jnp.dot(x_ref[...], w_ref[...],
                         preferred_element_type=f32) * dinv


def _tc2_body(a0_ref, a1_ref, d_ref, b_ref, w_ref, o_ref):
    dinv = _dinv(d_ref[...])
    h = (a0_ref[0] + a1_ref[0]) * dinv + b_ref[...]
    h = jnp.maximum(h, 0.0)
    o_ref[...] = jnp.dot(h, w_ref[...], preferred_element_type=f32) * dinv


def _tc3_body(a0_ref, a1_ref, d_ref, b_ref, o_ref):
    dinv = _dinv(d_ref[...])
    o_ref[...] = (a0_ref[0] + a1_ref[0]) * dinv + b_ref[...]


def _row_spec(width):
    return pl.BlockSpec((BLK, width), lambda i: (i, 0))


def _part_spec(core):
    return pl.BlockSpec((1, BLK, D), lambda i, c=core: (c, i, 0))


def _const_spec(shape):
    return pl.BlockSpec(shape, lambda i: (0,) * len(shape))


_tc1 = pl.pallas_call(
    _tc1_body,
    grid=(GRID,),
    in_specs=[_row_spec(D), _const_spec((D, D)), _row_spec(1)],
    out_specs=_row_spec(D),
    out_shape=jax.ShapeDtypeStruct((N, D), f32),
)

_tc2 = pl.pallas_call(
    _tc2_body,
    grid=(GRID,),
    in_specs=[_part_spec(0), _part_spec(1), _row_spec(1),
              _const_spec((1, D)), _const_spec((D, D))],
    out_specs=_row_spec(D),
    out_shape=jax.ShapeDtypeStruct((N, D), f32),
)

_tc3 = pl.pallas_call(
    _tc3_body,
    grid=(GRID,),
    in_specs=[_part_spec(0), _part_spec(1), _row_spec(1),
              _const_spec((1, D))],
    out_specs=_row_spec(D),
    out_shape=jax.ShapeDtypeStruct((N, D), f32),
)


def kernel(x, edge_index, W1, b1, W2, b2):
    ei3 = edge_index.astype(jnp.int32).reshape(2, RE, CHUNK)
    ei = jnp.concatenate([ei3, jnp.asarray(_PAD_NP)], axis=1)

    degp = _deg_kernel(ei)
    dsum = (degp[0] + degp[1])[:, None]

    b1r = b1.reshape(1, D)
    b2r = b2.reshape(1, D)

    t1 = _tc1(x, W1, dsum)
    p1 = _scatter_kernel(t1, ei)
    t2 = _tc2(p1, p1, dsum, b1r, W2)
    p2 = _scatter_kernel(t2, ei)
    out = _tc3(p2, p2, dsum, b2r)
    return out


# speedup vs baseline: 1.2513x; 1.2513x over previous
import functools

import jax
import jax.numpy as jnp
import numpy as np
from jax import lax
from jax.experimental import pallas as pl
from jax.experimental.pallas import tpu as pltpu
from jax.experimental.pallas import tpu_sc as plsc

N = 10000
E = 320000
D = 128

NC = 2
NS = 16
NW = NC * NS

CHUNK = 128
CHUNKS = 80
HALF = CHUNKS // 2
EPW = CHUNKS * CHUNK
E_PAD = EPW * NW

NPAD = 10240
RPS = NPAD // NS

RE = E // CHUNK
PAD_ROWS = NW * CHUNKS - RE

f32 = jnp.float32

_pi = np.arange(PAD_ROWS * CHUNK, dtype=np.int32)
_PAD_NP = np.stack([(_pi * 131) % N, N + (_pi % (NPAD - N))]).reshape(2, PAD_ROWS, CHUNK)


def _zero_vmem_2d(ref, rows, cols):
    assert cols % 16 == 0
    z = jnp.zeros((16,), f32)

    def body(r, _):
        for k in range(cols // 16):
            ref[r, pl.ds(16 * k, 16)] = z
        return 0

    lax.fori_loop(0, rows, body, 0)


@functools.partial(
    pl.kernel,
    out_type=jax.ShapeDtypeStruct((NC, NPAD), f32),
    mesh=plsc.VectorSubcoreMesh(core_axis_name="c", subcore_axis_name="s"),
    scratch_types=[
        pltpu.VMEM((CHUNKS, CHUNK), jnp.int32),
        pltpu.VMEM((CHUNK,), f32),
        pltpu.VMEM((CHUNK,), f32),
        pltpu.VMEM_SHARED((NPAD,), f32),
    ],
)
def _deg_kernel(ei_hbm, out_hbm, idx_v, ones_v, zeros_v, acc):
    c = lax.axis_index("c")
    s = lax.axis_index("s")
    wid = s * NC + c

    pltpu.sync_copy(ei_hbm.at[1, pl.ds(wid * CHUNKS, CHUNKS)], idx_v)

    one = jnp.ones((16,), f32)
    zero = jnp.zeros((16,), f32)
    for r in range(CHUNK // 16):
        ones_v[pl.ds(16 * r, 16)] = one
        zeros_v[pl.ds(16 * r, 16)] = zero

    base = s * RPS
    for k in range(RPS // CHUNK):
        pltpu.sync_copy(zeros_v, acc.at[pl.ds(base + k * CHUNK, CHUNK)])

    plsc.subcore_barrier()

    def step(j, _):
        pltpu.sync_copy(ones_v, acc.at[idx_v.at[j]], add=True)
        return 0

    lax.fori_loop(0, CHUNKS, step, 0)

    plsc.subcore_barrier()
    pltpu.sync_copy(acc.at[pl.ds(base, RPS)], out_hbm.at[c, pl.ds(base, RPS)])


@functools.partial(
    pl.kernel,
    out_type=jax.ShapeDtypeStruct((NC, NPAD, D), f32),
    mesh=plsc.VectorSubcoreMesh(core_axis_name="c", subcore_axis_name="s"),
    scratch_types=[
        pltpu.VMEM((HALF, CHUNK), jnp.int32),
        pltpu.VMEM((HALF, CHUNK), jnp.int32),
        pltpu.VMEM((CHUNK, D), f32),
        pltpu.VMEM((CHUNK, D), f32),
        pltpu.VMEM_SHARED((NPAD, D), f32),
        pltpu.SemaphoreType.DMA,
        pltpu.SemaphoreType.DMA,
    ],
)
def _scatter_kernel(t_hbm, ei_hbm, out_hbm,
                    src_v, dst_v, rows0_v, rows1_v, acc, sem0, sem1):
    c = lax.axis_index("c")
    s = lax.axis_index("s")
    wid = s * NC + c

    def stage(g):
        row0 = wid * CHUNKS + g * HALF
        pltpu.sync_copy(ei_hbm.at[0, pl.ds(row0, HALF)], src_v)
        pltpu.sync_copy(ei_hbm.at[1, pl.ds(row0, HALF)], dst_v)

    stage(0)
    pltpu.async_copy(t_hbm.at[src_v.at[0]], rows0_v, sem0)

    _zero_vmem_2d(rows1_v, CHUNK, D)

    base = s * RPS
    for k in range(RPS // CHUNK):
        pltpu.sync_copy(rows1_v, acc.at[pl.ds(base + k * CHUNK, CHUNK)])

    plsc.subcore_barrier()

    for g in range(2):
        if g:
            stage(1)
            pltpu.async_copy(t_hbm.at[src_v.at[0]], rows0_v, sem0)

        def step2(jj, _):
            j0 = 2 * jj
            j1 = j0 + 1

            pltpu.async_copy(t_hbm.at[src_v.at[j1]], rows1_v, sem1)

            pltpu.make_async_copy(t_hbm.at[src_v.at[j0]], rows0_v, sem0).wait()
            pltpu.sync_copy(rows0_v, acc.at[dst_v.at[j0]], add=True)

            @pl.when(j0 + 2 < HALF)
            def _():
                pltpu.async_copy(t_hbm.at[src_v.at[j0 + 2]], rows0_v, sem0)

            pltpu.make_async_copy(t_hbm.at[src_v.at[j1]], rows1_v, sem1).wait()
            pltpu.sync_copy(rows1_v, acc.at[dst_v.at[j1]], add=True)

            return 0

        lax.fori_loop(0, HALF // 2, step2, 0)

    plsc.subcore_barrier()
    pltpu.sync_copy(acc.at[pl.ds(base, RPS)], out_hbm.at[c, pl.ds(base, RPS)])


BLK = 2560
GRID = -(-N // BLK)


def _dinv(d):
    return lax.rsqrt(jnp.maximum(d, 1.0))


def _tc1_body(x_ref, w_ref, d_ref, o_ref):
    dinv = _dinv(d_ref[...])
    o_ref[...] = jnp.dot(x_ref[...], w_ref[...],
                         preferred_element_type=f32) * dinv


def _tc2_body(a0_ref, a1_ref, d_ref, b_ref, w_ref, o_ref):
    dinv = _dinv(d_ref[...])
    h = (a0_ref[0] + a1_ref[0]) * dinv + b_ref[...]
    h = jnp.maximum(h, 0.0)
    o_ref[...] = jnp.dot(h, w_ref[...], preferred_element_type=f32) * dinv


def _tc3_body(a0_ref, a1_ref, d_ref, b_ref, o_ref):
    dinv = _dinv(d_ref[...])
    o_ref[...] = (a0_ref[0] + a1_ref[0]) * dinv + b_ref[...]


def _row_spec(width):
    return pl.BlockSpec((BLK, width), lambda i: (i, 0))


def _part_spec(core):
    return pl.BlockSpec((1, BLK, D), lambda i, c=core: (c, i, 0))


def _const_spec(shape):
    return pl.BlockSpec(shape, lambda i: (0,) * len(shape))


_tc1 = pl.pallas_call(
    _tc1_body,
    grid=(GRID,),
    in_specs=[_row_spec(D), _const_spec((D, D)), _row_spec(1)],
    out_specs=_row_spec(D),
    out_shape=jax.ShapeDtypeStruct((N, D), f32),
)

_tc2 = pl.pallas_call(
    _tc2_body,
    grid=(GRID,),
    in_specs=[_part_spec(0), _part_spec(1), _row_spec(1),
              _const_spec((1, D)), _const_spec((D, D))],
    out_specs=_row_spec(D),
    out_shape=jax.ShapeDtypeStruct((N, D), f32),
)

_tc3 = pl.pallas_call(
    _tc3_body,
    grid=(GRID,),
    in_specs=[_part_spec(0), _part_spec(1), _row_spec(1),
              _const_spec((1, D))],
    out_specs=_row_spec(D),
    out_shape=jax.ShapeDtypeStruct((N, D), f32),
)


def kernel(x, edge_index, W1, b1, W2, b2):
    ei3 = edge_index.astype(jnp.int32).reshape(2, RE, CHUNK)
    ei = jnp.concatenate([ei3, jnp.asarray(_PAD_NP)], axis=1)

    degp = _deg_kernel(ei)
    dsum = (degp[0] + degp[1])[:, None]

    b1r = b1.reshape(1, D)
    b2r = b2.reshape(1, D)

    t1 = _tc1(x, W1, dsum)
    p1 = _scatter_kernel(t1, ei)
    t2 = _tc2(p1, p1, dsum, b1r, W2)
    p2 = _scatter_kernel(t2, ei)
    out = _tc3(p2, p2, dsum, b2r)
    return out
